# baseline (speedup 1.0000x reference)
import jax
import jax.numpy as jnp
import numpy as np
from jax.experimental import pallas as pl
from jax.experimental.pallas import tpu as pltpu

_NUM_CLASSES = 80
_SCORE_THR = 0.05
_IOU_THR = 0.5
_IMTOP = 100
_CANVAS_H = 800.0
_CANVAS_W = 1333.0
_CLIP = float(np.log(1000.0 / 16.0))

_BR = 256
_BC = 512


def _prep_body(inp_ref, stats_ref, boxes_ref):
    dx = inp_ref[0:1, :] * 0.1
    dy = inp_ref[1:2, :] * 0.1
    dw = jnp.minimum(inp_ref[2:3, :] * 0.2, _CLIP)
    dh = jnp.minimum(inp_ref[3:4, :] * 0.2, _CLIP)
    pw = inp_ref[6:7, :]
    ph = inp_ref[7:8, :]
    cx = inp_ref[4:5, :] + dx * pw
    cy = inp_ref[5:6, :] + dy * ph
    w = pw * jnp.exp(dw)
    h = ph * jnp.exp(dh)
    x1 = jnp.clip(cx - 0.5 * w, 0.0, _CANVAS_W)
    y1 = jnp.clip(cy - 0.5 * h, 0.0, _CANVAS_H)
    x2 = jnp.clip(cx + 0.5 * w, 0.0, _CANVAS_W)
    y2 = jnp.clip(cy + 0.5 * h, 0.0, _CANVAS_H)
    bw = x2 - x1
    bh = y2 - y1
    s = inp_ref[8:9, :]
    valid = (bw > 0.0) & (bh > 0.0) & (s > _SCORE_THR)
    off = inp_ref[9:10, :] * (_CANVAS_W + 1.0)
    npad = stats_ref.shape[1]
    stats_ref[0:1, :] = x1 + off
    stats_ref[1:2, :] = y1
    stats_ref[2:3, :] = x2 + off
    stats_ref[3:4, :] = y2
    stats_ref[4:5, :] = bw * bh
    stats_ref[5:6, :] = s
    stats_ref[6:7, :] = valid.astype(jnp.float32)
    stats_ref[7:8, :] = jax.lax.broadcasted_iota(jnp.int32, (1, npad), 1).astype(
        jnp.float32
    )
    boxes_ref[0:1, :] = x1
    boxes_ref[1:2, :] = y1
    boxes_ref[2:3, :] = x2
    boxes_ref[3:4, :] = y2


def _sup_body(statsI_ref, statsJ_ref, out_ref):
    j = pl.program_id(1)
    x1i = statsI_ref[:, 0:1]
    y1i = statsI_ref[:, 1:2]
    x2i = statsI_ref[:, 2:3]
    y2i = statsI_ref[:, 3:4]
    ai = statsI_ref[:, 4:5]
    si = statsI_ref[:, 5:6]
    ii = statsI_ref[:, 7:8]
    x1j = statsJ_ref[0:1, :]
    y1j = statsJ_ref[1:2, :]
    x2j = statsJ_ref[2:3, :]
    y2j = statsJ_ref[3:4, :]
    aj = statsJ_ref[4:5, :]
    sj = statsJ_ref[5:6, :]
    vj = statsJ_ref[6:7, :]
    ij = statsJ_ref[7:8, :]
    iw = jnp.maximum(jnp.minimum(x2i, x2j) - jnp.maximum(x1i, x1j), 0.0)
    ih = jnp.maximum(jnp.minimum(y2i, y2j) - jnp.maximum(y1i, y1j), 0.0)
    inter = iw * ih
    iou = inter / (ai + aj - inter + 1e-9)
    prec = (sj > si) | ((sj == si) & (ij < ii))
    hit = (iou > _IOU_THR) & prec & (vj > 0.5)
    sup = jnp.any(hit, axis=1, keepdims=True).astype(jnp.float32)

    @pl.when(j == 0)
    def _():
        out_ref[...] = jnp.zeros_like(out_ref)

    out_ref[:, 0:1] = jnp.maximum(out_ref[:, 0:1], sup)


def kernel(reg, proposals, scores, classes):
    n = reg.shape[0]
    npad = ((n + _BC - 1) // _BC) * _BC
    pad = npad - n
    inp = jnp.concatenate(
        [
            reg.T,
            proposals.T,
            scores[None, :],
            classes.astype(jnp.float32)[None, :],
            jnp.zeros((6, n), jnp.float32),
        ],
        axis=0,
    )
    inp = jnp.pad(inp, ((0, 0), (0, pad)), constant_values=-1.0)

    stats, boxesT = pl.pallas_call(
        _prep_body,
        out_shape=(
            jax.ShapeDtypeStruct((8, npad), jnp.float32),
            jax.ShapeDtypeStruct((4, npad), jnp.float32),
        ),
    )(inp)

    statsI = stats.T
    supout = pl.pallas_call(
        _sup_body,
        grid=(npad // _BR, npad // _BC),
        in_specs=[
            pl.BlockSpec((_BR, 8), lambda i, j: (i, 0)),
            pl.BlockSpec((8, _BC), lambda i, j: (0, j)),
        ],
        out_specs=pl.BlockSpec((_BR, 8), lambda i, j: (i, 0)),
        out_shape=jax.ShapeDtypeStruct((npad, 8), jnp.float32),
        compiler_params=pltpu.CompilerParams(
            dimension_semantics=("arbitrary", "arbitrary"),
        ),
    )(statsI, stats)

    suppressed = supout[:n, 0] > 0.5
    valid = stats[6, :n] > 0.5
    keep = valid & (~suppressed)
    vkey = jnp.where(valid, scores, -1.0)
    order = jnp.argsort(-vkey)
    keep_scores = jnp.where(keep, scores, -1.0)
    top_s, top_i = jax.lax.top_k(keep_scores[order], _IMTOP)
    sel = order[top_i]
    boxes = boxesT[:, :n].T
    return boxes[sel], scores[sel], classes[sel]
